# Initial kernel scaffold; baseline (speedup 1.0000x reference)
import jax
import jax.numpy as jnp
from jax.experimental import pallas as pl


def kernel(x, edge_index, W_self1, W_neigh1, b1, W_self2, W_neigh2, b2):
    raise NotImplementedError("write your pallas kernel here")


# speedup vs baseline: 7.2784x; 7.2784x over previous
import functools

import jax
import jax.numpy as jnp
from jax import lax
from jax.experimental import pallas as pl
from jax.experimental.pallas import tpu as pltpu
from jax.experimental.pallas import tpu_sc as plsc

N = 10000
D = 128
E = 320000
NC = 2
NS = 16
NW = NC * NS
EPW = E // NW
K = 80
NB = EPW // K
NP = N
RPT = NP // NS
DZ = 125


def _sc_agg_build(with_deg):
  mesh = plsc.VectorSubcoreMesh(core_axis_name="c", subcore_axis_name="s")
  out_type = [jax.ShapeDtypeStruct((NC, NP, D), jnp.float32)]
  scratch = [
      pltpu.VMEM((NB, K), jnp.int32),
      pltpu.VMEM((NB, K), jnp.int32),
      pltpu.VMEM((K, D), jnp.float32),
      pltpu.VMEM_SHARED((NP, D), jnp.float32),
      pltpu.SemaphoreType.DMA,
  ]
  if with_deg:
    out_type.append(jax.ShapeDtypeStruct((NC, NP, 16), jnp.float32))
    scratch += [
        pltpu.VMEM((K, 16), jnp.float32),
        pltpu.VMEM_SHARED((NP, 16), jnp.float32),
        pltpu.VMEM((DZ, 16), jnp.float32),
    ]

  def body(p_hbm, src_hbm, dst_hbm, *rest):
    if with_deg:
      (acc_o, deg_o, src_v, dst_v, rows_v, acc_sh, sem,
       ones_v, deg_sh, dzb_v) = rest
    else:
      acc_o, src_v, dst_v, rows_v, acc_sh, sem = rest
    cid = lax.axis_index("c")
    sid = lax.axis_index("s")
    wid = cid * NS + sid

    def zfill(i, _):
      for c in range(D // 16):
        rows_v[i, pl.ds(c * 16, 16)] = jnp.zeros((16,), jnp.float32)
      return 0
    lax.fori_loop(0, K, zfill, 0)
    for j in range(RPT // K):
      pltpu.sync_copy(rows_v, acc_sh.at[pl.ds(sid * RPT + j * K, K)])
    rem = RPT % K
    if rem:
      pltpu.sync_copy(rows_v.at[pl.ds(0, rem)],
                      acc_sh.at[pl.ds(sid * RPT + (RPT // K) * K, rem)])
    if with_deg:
      def dzfill(i, _):
        dzb_v[i, :] = jnp.zeros((16,), jnp.float32)
        return 0
      lax.fori_loop(0, DZ, dzfill, 0)
      for j in range(RPT // DZ):
        pltpu.sync_copy(dzb_v, deg_sh.at[pl.ds(sid * RPT + j * DZ, DZ)])
      def ofill(i, _):
        ones_v[i, :] = jnp.ones((16,), jnp.float32)
        return 0
      lax.fori_loop(0, K, ofill, 0)
    plsc.subcore_barrier()

    pltpu.sync_copy(src_hbm.at[wid], src_v)
    pltpu.sync_copy(dst_hbm.at[wid], dst_v)

    def step(i, _):
      pltpu.async_copy(p_hbm.at[src_v.at[i]], rows_v, sem).wait()
      pltpu.sync_copy(rows_v, acc_sh.at[dst_v.at[i]], add=True)
      if with_deg:
        pltpu.sync_copy(ones_v, deg_sh.at[dst_v.at[i]], add=True)
      return 0
    lax.fori_loop(0, NB, step, 0)
    plsc.subcore_barrier()

    r0 = sid * RPT
    pltpu.sync_copy(acc_sh.at[pl.ds(r0, RPT)],
                    acc_o.at[cid, pl.ds(r0, RPT)])
    if with_deg:
      pltpu.sync_copy(deg_sh.at[pl.ds(r0, RPT)],
                      deg_o.at[cid, pl.ds(r0, RPT)])

  return pl.kernel(
      body, out_type=out_type, mesh=mesh, scratch_types=scratch,
      compiler_params=pltpu.CompilerParams(use_tc_tiling_on_sc=False))


_sc_agg_deg = _sc_agg_build(True)
_sc_agg = _sc_agg_build(False)


def _dot(a, b):
  return jnp.dot(a, b, preferred_element_type=jnp.float32,
                 precision=lax.Precision.HIGHEST)


def _layer1_body(x_ref, a0_ref, a1_ref, d0_ref, d1_ref, ws_ref, wn_ref,
                 b_ref, o_ref, inv_ref):
  deg = jnp.sum(d0_ref[...] + d1_ref[...], axis=1, keepdims=True) * (1.0 / 16.0)
  inv = 1.0 / jnp.maximum(deg, 1.0)
  agg = (a0_ref[...] + a1_ref[...]) * inv
  y = _dot(x_ref[...], ws_ref[...]) + _dot(agg, wn_ref[...]) + b_ref[...]
  o_ref[...] = jnp.maximum(y, 0.0)
  inv_ref[...] = inv


def _layer2_body(h_ref, a0_ref, a1_ref, inv_ref, ws_ref, wn_ref, b_ref,
                 o_ref):
  agg = (a0_ref[...] + a1_ref[...]) * inv_ref[...]
  y = _dot(h_ref[...], ws_ref[...]) + _dot(agg, wn_ref[...]) + b_ref[...]
  y = jnp.maximum(y, 0.0)
  nrm = jnp.sqrt(jnp.sum(y * y, axis=1, keepdims=True))
  o_ref[...] = y / jnp.maximum(nrm, 1e-12)


_BR = 1000
_row = pl.BlockSpec((_BR, D), lambda i: (i, 0))
_col1 = pl.BlockSpec((_BR, 1), lambda i: (i, 0))
_col16 = pl.BlockSpec((_BR, 16), lambda i: (i, 0))
_wspec = pl.BlockSpec((D, D), lambda i: (0, 0))
_bspec = pl.BlockSpec((1, D), lambda i: (0, 0))

_tc_layer1 = pl.pallas_call(
    _layer1_body,
    grid=(N // _BR,),
    in_specs=[_row, _row, _row, _col16, _col16, _wspec, _wspec, _bspec],
    out_specs=[_row, _col1],
    out_shape=[jax.ShapeDtypeStruct((N, D), jnp.float32),
               jax.ShapeDtypeStruct((N, 1), jnp.float32)],
)

_tc_layer2 = pl.pallas_call(
    _layer2_body,
    grid=(N // _BR,),
    in_specs=[_row, _row, _row, _col1, _wspec, _wspec, _bspec],
    out_specs=_row,
    out_shape=jax.ShapeDtypeStruct((N, D), jnp.float32),
)


@jax.jit
def kernel(x, edge_index, W_self1, W_neigh1, b1, W_self2, W_neigh2, b2):
  src = edge_index[0].astype(jnp.int32).reshape(NW, NB, K)
  dst = edge_index[1].astype(jnp.int32).reshape(NW, NB, K)
  acc1, degp = _sc_agg_deg(x, src, dst)
  h1, inv = _tc_layer1(x, acc1[0, :N], acc1[1, :N], degp[0, :N], degp[1, :N],
                       W_self1, W_neigh1, b1.reshape(1, D))
  (acc2,) = _sc_agg(h1, src, dst)
  return _tc_layer2(h1, acc2[0, :N], acc2[1, :N], inv,
                    W_self2, W_neigh2, b2.reshape(1, D))


# speedup vs baseline: 9.4087x; 1.2927x over previous
import functools

import jax
import jax.numpy as jnp
from jax import lax
from jax.experimental import pallas as pl
from jax.experimental.pallas import tpu as pltpu
from jax.experimental.pallas import tpu_sc as plsc

N = 10000
D = 128
E = 320000
NC = 2
NS = 16
NW = NC * NS
EPW = E // NW
K = 40
NB = EPW // K
NP = N
RPT = NP // NS
DZ = 25


def _sc_agg_build(with_deg):
  mesh = plsc.VectorSubcoreMesh(core_axis_name="c", subcore_axis_name="s")
  out_type = [jax.ShapeDtypeStruct((NC, NP, D), jnp.float32)]
  scratch = [
      pltpu.VMEM((NB, K), jnp.int32),
      pltpu.VMEM((NB, K), jnp.int32),
      pltpu.VMEM((K, D), jnp.float32),
      pltpu.VMEM((K, D), jnp.float32),
      pltpu.VMEM_SHARED((NP, D), jnp.float32),
      pltpu.SemaphoreType.DMA,
      pltpu.SemaphoreType.DMA,
  ]
  if with_deg:
    out_type.append(jax.ShapeDtypeStruct((NC, NP, 16), jnp.float32))
    scratch += [
        pltpu.VMEM((K, 16), jnp.float32),
        pltpu.VMEM_SHARED((NP, 16), jnp.float32),
        pltpu.VMEM((DZ, 16), jnp.float32),
    ]

  def body(p_hbm, src_hbm, dst_hbm, *rest):
    if with_deg:
      (acc_o, deg_o, src_v, dst_v, rows_v, rows2_v, acc_sh, sem, sem2,
       ones_v, deg_sh, dzb_v) = rest
    else:
      acc_o, src_v, dst_v, rows_v, rows2_v, acc_sh, sem, sem2 = rest
    cid = lax.axis_index("c")
    sid = lax.axis_index("s")
    wid = cid * NS + sid

    def zfill(i, _):
      for c in range(D // 16):
        rows_v[i, pl.ds(c * 16, 16)] = jnp.zeros((16,), jnp.float32)
      return 0
    lax.fori_loop(0, K, zfill, 0)
    for j in range(RPT // K):
      pltpu.sync_copy(rows_v, acc_sh.at[pl.ds(sid * RPT + j * K, K)])
    rem = RPT % K
    if rem:
      pltpu.sync_copy(rows_v.at[pl.ds(0, rem)],
                      acc_sh.at[pl.ds(sid * RPT + (RPT // K) * K, rem)])
    if with_deg:
      def dzfill(i, _):
        dzb_v[i, :] = jnp.zeros((16,), jnp.float32)
        return 0
      lax.fori_loop(0, DZ, dzfill, 0)
      for j in range(RPT // DZ):
        pltpu.sync_copy(dzb_v, deg_sh.at[pl.ds(sid * RPT + j * DZ, DZ)])
      def ofill(i, _):
        ones_v[i, :] = jnp.ones((16,), jnp.float32)
        return 0
      lax.fori_loop(0, K, ofill, 0)
    plsc.subcore_barrier()

    pltpu.sync_copy(src_hbm.at[wid], src_v)
    pltpu.sync_copy(dst_hbm.at[wid], dst_v)

    bufs = ((rows_v, sem), (rows2_v, sem2))

    def start(i, b):
      pltpu.async_copy(p_hbm.at[src_v.at[i]], bufs[b][0], bufs[b][1])

    def finish(i, b):
      pltpu.make_async_copy(p_hbm.at[src_v.at[0]], bufs[b][0],
                            bufs[b][1]).wait()
      pltpu.sync_copy(bufs[b][0], acc_sh.at[dst_v.at[i]], add=True)
      if with_deg:
        pltpu.sync_copy(ones_v, deg_sh.at[dst_v.at[i]], add=True)

    start(0, 0)
    start(1, 1)

    def step(g, _):
      i0 = 2 * g
      finish(i0, 0)
      start(i0 + 2, 0)
      finish(i0 + 1, 1)
      start(i0 + 3, 1)
      return 0
    lax.fori_loop(0, (NB - 2) // 2, step, 0)
    finish(NB - 2, 0)
    finish(NB - 1, 1)
    plsc.subcore_barrier()

    r0 = sid * RPT
    pltpu.sync_copy(acc_sh.at[pl.ds(r0, RPT)],
                    acc_o.at[cid, pl.ds(r0, RPT)])
    if with_deg:
      pltpu.sync_copy(deg_sh.at[pl.ds(r0, RPT)],
                      deg_o.at[cid, pl.ds(r0, RPT)])

  return pl.kernel(
      body, out_type=out_type, mesh=mesh, scratch_types=scratch,
      compiler_params=pltpu.CompilerParams(use_tc_tiling_on_sc=False))


_sc_agg_deg = _sc_agg_build(True)
_sc_agg = _sc_agg_build(False)


def _dot(a, b):
  return jnp.dot(a, b, preferred_element_type=jnp.float32,
                 precision=lax.Precision.HIGHEST)


def _layer1_body(x_ref, a0_ref, a1_ref, deg_ref, ws_ref, wn_ref,
                 b_ref, o_ref, inv_ref):
  deg = jnp.sum(deg_ref[0] + deg_ref[1], axis=1, keepdims=True) * (1.0 / 16.0)
  inv = 1.0 / jnp.maximum(deg, 1.0)
  agg = (a0_ref[0] + a1_ref[0]) * inv
  y = _dot(x_ref[...], ws_ref[...]) + _dot(agg, wn_ref[...]) + b_ref[...]
  o_ref[...] = jnp.maximum(y, 0.0)
  inv_ref[...] = inv


def _layer2_body(h_ref, a0_ref, a1_ref, inv_ref, ws_ref, wn_ref, b_ref,
                 o_ref):
  agg = (a0_ref[0] + a1_ref[0]) * inv_ref[...]
  y = _dot(h_ref[...], ws_ref[...]) + _dot(agg, wn_ref[...]) + b_ref[...]
  y = jnp.maximum(y, 0.0)
  nrm = jnp.sqrt(jnp.sum(y * y, axis=1, keepdims=True))
  o_ref[...] = y / jnp.maximum(nrm, 1e-12)


_BR = 1000
_row = pl.BlockSpec((_BR, D), lambda i: (i, 0))
_col1 = pl.BlockSpec((_BR, 1), lambda i: (i, 0))
_acc0 = pl.BlockSpec((1, _BR, D), lambda i: (0, i, 0))
_acc1 = pl.BlockSpec((1, _BR, D), lambda i: (1, i, 0))
_degs = pl.BlockSpec((2, _BR, 16), lambda i: (0, i, 0))
_wspec = pl.BlockSpec((D, D), lambda i: (0, 0))
_bspec = pl.BlockSpec((1, D), lambda i: (0, 0))

_tc_layer1 = pl.pallas_call(
    _layer1_body,
    grid=(N // _BR,),
    in_specs=[_row, _acc0, _acc1, _degs, _wspec, _wspec, _bspec],
    out_specs=[_row, _col1],
    out_shape=[jax.ShapeDtypeStruct((N, D), jnp.float32),
               jax.ShapeDtypeStruct((N, 1), jnp.float32)],
)

_tc_layer2 = pl.pallas_call(
    _layer2_body,
    grid=(N // _BR,),
    in_specs=[_row, _acc0, _acc1, _col1, _wspec, _wspec, _bspec],
    out_specs=_row,
    out_shape=jax.ShapeDtypeStruct((N, D), jnp.float32),
)


@jax.jit
def kernel(x, edge_index, W_self1, W_neigh1, b1, W_self2, W_neigh2, b2):
  src = edge_index[0].astype(jnp.int32).reshape(NW, NB, K)
  dst = edge_index[1].astype(jnp.int32).reshape(NW, NB, K)
  acc1, degp = _sc_agg_deg(x, src, dst)
  h1, inv = _tc_layer1(x, acc1, acc1, degp,
                       W_self1, W_neigh1, b1.reshape(1, D))
  (acc2,) = _sc_agg(h1, src, dst)
  return _tc_layer2(h1, acc2, acc2, inv,
                    W_self2, W_neigh2, b2.reshape(1, D))
